# Initial kernel scaffold; baseline (speedup 1.0000x reference)
import jax
import jax.numpy as jnp
from jax.experimental import pallas as pl


def kernel(features, edge_index, edge_types, W1, attn_l1, attn_r1, bias1, W2, attn_l2, attn_r2, bias2, Wp, bp):
    raise NotImplementedError("write your pallas kernel here")


# speedup vs baseline: 16.4567x; 16.4567x over previous
import functools

import jax
import jax.numpy as jnp
from jax import lax
from jax.experimental import pallas as pl
from jax.experimental.pallas import tpu as pltpu
from jax.experimental.pallas import tpu_sc as plsc

N = 10000
E = 320000
NP = 10240
HALF = NP // 2
HR = HALF // 16
AROWS = HALF + 8
C = 64
CW = 316
EP = 16 * CW * C
W2ACC = 80

_f32 = jnp.float32
_i32 = jnp.int32


def _mesh():
    return plsc.VectorSubcoreMesh(core_axis_name="c", subcore_axis_name="s")


def _localize(didx, dloc, c):
    lo = c * HALF
    for k in range(C // 16):
        d16 = didx[pl.ds(16 * k, 16)]
        loc = d16 - lo
        ok = (loc >= 0) & (loc < HALF)
        dloc[pl.ds(16 * k, 16)] = jnp.where(ok, loc, HALF)


def _zero_buf(buf, width):
    z16 = jnp.zeros((16,), _f32)

    def _zero(j, _):
        for k in range(width // 16):
            buf[j, pl.ds(16 * k, 16)] = z16
        return 0

    lax.fori_loop(0, 64, _zero, 0)


def _zero_acc(acc, buf, s):
    a0 = s * HR
    for p in range(5):
        pltpu.sync_copy(buf.at[pl.ds(0, 64)], acc.at[pl.ds(a0 + p * 64, 64)])

    @pl.when(s == 0)
    def _():
        pltpu.sync_copy(buf.at[pl.ds(0, 8)], acc.at[pl.ds(HALF, 8)])


def _writeback(acc, buf, out_hbm, c, s):
    a0 = s * HR
    g0 = c * HALF + s * HR
    for p in range(5):
        pltpu.sync_copy(acc.at[pl.ds(a0 + p * 64, 64)], buf.at[pl.ds(0, 64)])
        pltpu.sync_copy(buf.at[pl.ds(0, 64)], out_hbm.at[pl.ds(g0 + p * 64, 64)])


def _sc_edges1(src_hbm, dst_hbm, x_hbm, er_hbm, out_hbm, den_hbm,
               sidx, didx, dloc, xrows, arows, exb, mrows,
               acc, dacc, sem_x, sem_a):
    c = lax.axis_index("c")
    s = lax.axis_index("s")

    _zero_buf(mrows, 128)
    _zero_buf(exb, 16)
    _zero_acc(acc, mrows, s)
    _zero_acc(dacc, exb, s)
    plsc.subcore_barrier()

    def _chunk(g, _):
        base = (s * CW + g) * C
        pltpu.sync_copy(src_hbm.at[pl.ds(base, C)], sidx)
        pltpu.sync_copy(dst_hbm.at[pl.ds(base, C)], didx)
        cpx = pltpu.async_copy(x_hbm.at[sidx], xrows, sem_x)
        cpa = pltpu.async_copy(er_hbm.at[didx], arows, sem_a)
        _localize(didx, dloc, c)
        cpx.wait()
        cpa.wait()

        def _edge(j, _):
            for u in range(4):
                jj = j * 4 + u
                ev = xrows[jj, pl.ds(128, 16)] + arows[jj, pl.ds(0, 16)]
                ev = jnp.maximum(ev, 0.2 * ev)
                ex = jnp.exp(ev)
                exb[jj, :] = ex
                for h in range(8):
                    mrows[jj, pl.ds(16 * h, 16)] = (
                        xrows[jj, pl.ds(16 * h, 16)] * ex[h])
            return 0

        lax.fori_loop(0, C // 4, _edge, 0)
        pltpu.sync_copy(mrows, acc.at[dloc], add=True)
        pltpu.sync_copy(exb, dacc.at[dloc], add=True)
        return 0

    lax.fori_loop(0, CW, _chunk, 0)
    plsc.subcore_barrier()
    _writeback(acc, mrows, out_hbm, c, s)
    _writeback(dacc, exb, den_hbm, c, s)


def _sc1():
  return pl.kernel(
    _sc_edges1,
    out_type=[jax.ShapeDtypeStruct((NP, 128), _f32),
              jax.ShapeDtypeStruct((NP, 16), _f32)],
    mesh=_mesh(),
    scratch_types=[
        pltpu.VMEM((C,), _i32),
        pltpu.VMEM((C,), _i32),
        pltpu.VMEM((C,), _i32),
        pltpu.VMEM((C, 256), _f32),
        pltpu.VMEM((C, 128), _f32),
        pltpu.VMEM((C, 16), _f32),
        pltpu.VMEM((C, 128), _f32),
        pltpu.VMEM_SHARED((AROWS, 128), _f32),
        pltpu.VMEM_SHARED((AROWS, 16), _f32),
        pltpu.SemaphoreType.DMA,
        pltpu.SemaphoreType.DMA,
    ],
  )


def _sc_edges2(src_hbm, dst_hbm, x_hbm, er_hbm, out_hbm,
               sidx, didx, dloc, xrows, arows, mrows,
               acc, sem_x, sem_a):
    c = lax.axis_index("c")
    s = lax.axis_index("s")

    _zero_buf(mrows, W2ACC)
    _zero_acc(acc, mrows, s)
    plsc.subcore_barrier()

    def _chunk(g, _):
        base = (s * CW + g) * C
        pltpu.sync_copy(src_hbm.at[pl.ds(base, C)], sidx)
        pltpu.sync_copy(dst_hbm.at[pl.ds(base, C)], didx)
        cpx = pltpu.async_copy(x_hbm.at[sidx], xrows, sem_x)
        cpa = pltpu.async_copy(er_hbm.at[didx], arows, sem_a)
        _localize(didx, dloc, c)
        cpx.wait()
        cpa.wait()

        def _edge(j, _):
            for u in range(4):
                jj = j * 4 + u
                ev = xrows[jj, pl.ds(64, 16)] + arows[jj, pl.ds(0, 16)]
                ev = jnp.maximum(ev, 0.2 * ev)
                ex = jnp.exp(ev)
                mrows[jj, pl.ds(64, 16)] = ex
                for k in range(4):
                    mrows[jj, pl.ds(16 * k, 16)] = (
                        xrows[jj, pl.ds(16 * k, 16)] * ex[0])
            return 0

        lax.fori_loop(0, C // 4, _edge, 0)
        pltpu.sync_copy(mrows, acc.at[dloc], add=True)
        return 0

    lax.fori_loop(0, CW, _chunk, 0)
    plsc.subcore_barrier()
    _writeback(acc, mrows, out_hbm, c, s)


def _sc2():
  return pl.kernel(
    _sc_edges2,
    out_type=jax.ShapeDtypeStruct((NP, W2ACC), _f32),
    mesh=_mesh(),
    scratch_types=[
        pltpu.VMEM((C,), _i32),
        pltpu.VMEM((C,), _i32),
        pltpu.VMEM((C,), _i32),
        pltpu.VMEM((C, 128), _f32),
        pltpu.VMEM((C, 128), _f32),
        pltpu.VMEM((C, W2ACC), _f32),
        pltpu.VMEM_SHARED((AROWS, W2ACC), _f32),
        pltpu.SemaphoreType.DMA,
        pltpu.SemaphoreType.DMA,
    ],
  )


_BT = 2048


def _tc_a_body(x_ref, w1_ref, ela_ref, era_ref, xo_ref, er_ref):
    f = jnp.dot(x_ref[...], w1_ref[...], preferred_element_type=_f32)
    el = jnp.dot(f, ela_ref[...], preferred_element_type=_f32)
    z = jnp.zeros((f.shape[0], 112), _f32)
    xo_ref[...] = jnp.concatenate([f, el, z], axis=1)
    er_ref[...] = jnp.dot(f, era_ref[...], preferred_element_type=_f32)


def _tc_a(xp, W1, ela, era128):
    return pl.pallas_call(
        _tc_a_body,
        grid=(NP // _BT,),
        in_specs=[
            pl.BlockSpec((_BT, 128), lambda i: (i, 0)),
            pl.BlockSpec((128, 128), lambda i: (0, 0)),
            pl.BlockSpec((128, 16), lambda i: (0, 0)),
            pl.BlockSpec((128, 128), lambda i: (0, 0)),
        ],
        out_specs=[
            pl.BlockSpec((_BT, 256), lambda i: (i, 0)),
            pl.BlockSpec((_BT, 128), lambda i: (i, 0)),
        ],
        out_shape=[
            jax.ShapeDtypeStruct((NP, 256), _f32),
            jax.ShapeDtypeStruct((NP, 128), _f32),
        ],
    )(xp, W1, ela, era128)


def _tc_b_body(q_ref, d_ref, b1_ref, exp_ref, w2x_ref, wr2_ref,
               x2_ref, er2_ref):
    den = jnp.dot(d_ref[...], exp_ref[...],
                  preferred_element_type=_f32) + 1e-16
    x2 = jnp.maximum(q_ref[...] / den + b1_ref[...], 0.0)
    x2_ref[...] = jnp.dot(x2, w2x_ref[...], preferred_element_type=_f32)
    er2_ref[...] = jnp.dot(x2, wr2_ref[...], preferred_element_type=_f32)


def _tc_b(q, d, b1, exp16, W2x, Wr2):
    return pl.pallas_call(
        _tc_b_body,
        grid=(NP // _BT,),
        in_specs=[
            pl.BlockSpec((_BT, 128), lambda i: (i, 0)),
            pl.BlockSpec((_BT, 16), lambda i: (i, 0)),
            pl.BlockSpec((1, 128), lambda i: (0, 0)),
            pl.BlockSpec((16, 128), lambda i: (0, 0)),
            pl.BlockSpec((128, 128), lambda i: (0, 0)),
            pl.BlockSpec((128, 128), lambda i: (0, 0)),
        ],
        out_specs=[
            pl.BlockSpec((_BT, 128), lambda i: (i, 0)),
            pl.BlockSpec((_BT, 128), lambda i: (i, 0)),
        ],
        out_shape=[
            jax.ShapeDtypeStruct((NP, 128), _f32),
            jax.ShapeDtypeStruct((NP, 128), _f32),
        ],
    )(q, d, b1, exp16, W2x, Wr2)


def _tc_c_body(q_ref, b2_ref, wp_ref, bp_ref, y_ref):
    q = q_ref[...]
    den = q[:, 64:65] + 1e-16
    x3 = jnp.maximum(q[:, :64] / den + b2_ref[...], 0.0)
    y = jnp.dot(x3, wp_ref[...], preferred_element_type=_f32) + bp_ref[...]
    y_ref[...] = 1.0 / (1.0 + jnp.exp(-y))


def _tc_c(q, b2, wp8, bp8):
    return pl.pallas_call(
        _tc_c_body,
        grid=(NP // _BT,),
        in_specs=[
            pl.BlockSpec((_BT, W2ACC), lambda i: (i, 0)),
            pl.BlockSpec((1, 64), lambda i: (0, 0)),
            pl.BlockSpec((64, 8), lambda i: (0, 0)),
            pl.BlockSpec((1, 8), lambda i: (0, 0)),
        ],
        out_specs=pl.BlockSpec((_BT, 8), lambda i: (i, 0)),
        out_shape=jax.ShapeDtypeStruct((NP, 8), _f32),
    )(q, b2, wp8, bp8)


def kernel(features, edge_index, edge_types, W1, attn_l1, attn_r1, bias1,
           W2, attn_l2, attn_r2, bias2, Wp, bp):
    del edge_types

    xp = jnp.pad(features, ((0, NP - N), (0, 0)))
    srcp = jnp.concatenate(
        [edge_index[0], jnp.full((EP - E,), N, _i32)])
    dstp = jnp.concatenate(
        [edge_index[1], jnp.full((EP - E,), N, _i32)])

    eye8_16 = jnp.concatenate([jnp.eye(8, dtype=_f32),
                               jnp.zeros((8, 8), _f32)], axis=1)
    ela = (attn_l1[0][:, :, None] * eye8_16[:, None, :]).reshape(128, 16)
    era = (attn_r1[0][:, :, None] * eye8_16[:, None, :]).reshape(128, 16)
    era128 = jnp.pad(era, ((0, 0), (0, 112)))
    exp16 = jnp.concatenate(
        [jnp.kron(jnp.eye(8, dtype=_f32), jnp.ones((1, 16), _f32)),
         jnp.zeros((8, 128), _f32)], axis=0)
    el2w = W2 @ attn_l2[0, 0]
    er2w = W2 @ attn_r2[0, 0]
    W2x = jnp.concatenate(
        [W2, el2w[:, None], jnp.zeros((128, 63), _f32)], axis=1)
    Wr2 = jnp.concatenate(
        [er2w[:, None], jnp.zeros((128, 127), _f32)], axis=1)
    wp8 = jnp.zeros((64, 8), _f32).at[:, 0].set(Wp[:, 0])
    bp8 = jnp.broadcast_to(bp[0], (1, 8))
    b1 = bias1.reshape(1, 128)
    b2 = bias2.reshape(1, 64)

    x1, er1 = _tc_a(xp, W1, ela, era128)
    out1, den1 = _sc1()(srcp, dstp, x1, er1)

    x2, er2 = _tc_b(out1, den1, b1, exp16, W2x, Wr2)
    out2 = _sc2()(srcp, dstp, x2, er2)

    y8 = _tc_c(out2, b2, wp8, bp8)
    return y8[:N, 0]
